# baseline (speedup 1.0000x reference)
import jax
import jax.numpy as jnp
from jax.experimental import pallas as pl
from jax.experimental.pallas import tpu as pltpu

_NMS_PRE = 2048
_NMS_POST = 512
_NMS_THRESH = 0.7
_NUM_CLASS = 3
_SUB = 16
_LANE = 128


def _nms_body(abv_ref, absm_ref, pk_ref,
              rois_ref, scr_ref, labo_ref, lgo_ref,
              kps):
    kps[...] = jnp.ones_like(kps)
    x1v = abv_ref[0, 0]
    y1v = abv_ref[0, 1]
    x2v = abv_ref[0, 2]
    y2v = abv_ref[0, 3]
    arv = abv_ref[0, 4]

    idx = (jax.lax.broadcasted_iota(jnp.int32, (_SUB, _LANE), 0) * _LANE
           + jax.lax.broadcasted_iota(jnp.int32, (_SUB, _LANE), 1))
    lanes = jax.lax.broadcasted_iota(jnp.int32, (1, _LANE), 1)

    def cond(carry):
        i, count, _ = carry
        return (i < _NMS_PRE) & (count < _NMS_POST)

    def body(carry):
        i, count, dstv = carry
        oneb = idx == i
        row = kps[pl.ds(i // _LANE, 1), :]
        keep_i = jnp.sum(row * (lanes == i % _LANE).astype(jnp.float32))
        x1i = absm_ref[0, 0, i]
        y1i = absm_ref[0, 1, i]
        x2i = absm_ref[0, 2, i]
        y2i = absm_ref[0, 3, i]
        ai = absm_ref[0, 4, i]
        ix1 = jnp.maximum(x1v, x1i)
        iy1 = jnp.maximum(y1v, y1i)
        ix2 = jnp.minimum(x2v, x2i)
        iy2 = jnp.minimum(y2v, y2i)
        inter = jnp.maximum(ix2 - ix1, 0.0) * jnp.maximum(iy2 - iy1, 0.0)
        iou = inter / (ai + arv - inter + 1e-8)
        alive = keep_i > 0.0
        sup = (iou > _NMS_THRESH) & alive
        kps[...] = jnp.where(sup, 0.0, kps[...])
        dstv = jnp.where(oneb & alive, count, dstv)
        return i + 1, count + alive.astype(jnp.int32), dstv

    _, _, dstv = jax.lax.while_loop(
        cond, body,
        (jnp.int32(0), jnp.int32(0),
         jnp.full((_SUB, _LANE), _NMS_POST, jnp.int32)))

    slot = jax.lax.broadcasted_iota(jnp.int32, (_NMS_POST, _LANE), 0)
    out = jnp.zeros((_NMS_POST, 16), jnp.float32)
    for k in range(_SUB):
        sel = (slot == jnp.broadcast_to(
            dstv[k:k + 1, :], (_NMS_POST, _LANE))).astype(jnp.float32)
        out = out + jnp.dot(sel, pk_ref[0, k * _LANE:(k + 1) * _LANE, :],
                            preferred_element_type=jnp.float32)
    rois_ref[0] = out[:, 0:7]
    scr_ref[0] = out[:, 7:8]
    labo_ref[0] = out[:, 11:12].astype(jnp.int32)
    lgo_ref[0] = out[:, 8:11]


@jax.jit
def _run(abv, absm, pk):
    B = abv.shape[0]
    return pl.pallas_call(
        _nms_body,
        grid=(B,),
        in_specs=[
            pl.BlockSpec((1, 5, _SUB, _LANE), lambda b: (b, 0, 0, 0)),
            pl.BlockSpec((1, 5, _NMS_PRE), lambda b: (b, 0, 0),
                         memory_space=pltpu.SMEM),
            pl.BlockSpec((1, _NMS_PRE, 16), lambda b: (b, 0, 0)),
        ],
        out_specs=[
            pl.BlockSpec((1, _NMS_POST, 7), lambda b: (b, 0, 0)),
            pl.BlockSpec((1, _NMS_POST, 1), lambda b: (b, 0, 0)),
            pl.BlockSpec((1, _NMS_POST, 1), lambda b: (b, 0, 0)),
            pl.BlockSpec((1, _NMS_POST, _NUM_CLASS), lambda b: (b, 0, 0)),
        ],
        out_shape=[
            jax.ShapeDtypeStruct((B, _NMS_POST, 7), jnp.float32),
            jax.ShapeDtypeStruct((B, _NMS_POST, 1), jnp.float32),
            jax.ShapeDtypeStruct((B, _NMS_POST, 1), jnp.int32),
            jax.ShapeDtypeStruct((B, _NMS_POST, _NUM_CLASS), jnp.float32),
        ],
        scratch_shapes=[pltpu.VMEM((_SUB, _LANE), jnp.float32)],
        compiler_params=pltpu.CompilerParams(
            dimension_semantics=("parallel",)),
    )(abv, absm, pk)


def kernel(batch_box_preds, batch_cls_preds, batch_size):
    scores = jnp.max(batch_cls_preds, axis=-1)
    _, order = jax.lax.top_k(scores, _NMS_PRE)
    payload = jnp.concatenate([batch_box_preds, batch_cls_preds], axis=-1)
    g = jnp.take_along_axis(payload, order[..., None], axis=1)
    b = g[..., 0:7]
    lg = g[..., 7:10]
    top_s = jnp.max(lg, axis=-1)
    lab = jnp.argmax(lg, axis=-1).astype(jnp.int32)
    B = b.shape[0]
    x = b[..., 0]
    y = b[..., 1]
    dx = b[..., 3]
    dy = b[..., 4]
    ry = b[..., 6]
    c = jnp.abs(jnp.cos(ry))
    s = jnp.abs(jnp.sin(ry))
    hw = 0.5 * (dx * c + dy * s)
    hh = 0.5 * (dx * s + dy * c)
    x1 = x - hw
    y1 = y - hh
    x2 = x + hw
    y2 = y + hh
    ab5 = jnp.stack([x1, y1, x2, y2, (x2 - x1) * (y2 - y1)], axis=1)
    abv = ab5.reshape(B, 5, _SUB, _LANE)
    pk = jnp.concatenate(
        [b, top_s[..., None], lg, (lab + 1).astype(jnp.float32)[..., None],
         jnp.zeros((B, _NMS_PRE, 4), jnp.float32)], axis=-1)
    rois, scr, labo, lgo = _run(abv, ab5, pk)
    return rois, scr[:, :, 0], labo[:, :, 0], lgo


# speedup vs baseline: 1.2991x; 1.2991x over previous
import jax
import jax.numpy as jnp
from jax.experimental import pallas as pl
from jax.experimental.pallas import tpu as pltpu

_NMS_PRE = 2048
_NMS_POST = 512
_NMS_THRESH = 0.7
_NUM_CLASS = 3
_SUB = 16
_LANE = 128


def _nms_body(abv_ref, absm_ref, pk_ref,
              rois_ref, scr_ref, labo_ref, lgo_ref,
              kps):
    kps[...] = jnp.ones_like(kps)
    x1v = abv_ref[0, 0]
    y1v = abv_ref[0, 1]
    x2v = abv_ref[0, 2]
    y2v = abv_ref[0, 3]
    arv = abv_ref[0, 4]

    idx = (jax.lax.broadcasted_iota(jnp.int32, (_SUB, _LANE), 0) * _LANE
           + jax.lax.broadcasted_iota(jnp.int32, (_SUB, _LANE), 1))
    lanes = jax.lax.broadcasted_iota(jnp.int32, (1, _LANE), 1)

    def cond(carry):
        i, count, _ = carry
        return (i < _NMS_PRE) & (count < _NMS_POST)

    def body(carry):
        i, count, dstv = carry
        i1 = i + 1
        row0 = kps[pl.ds(i // _LANE, 1), :]
        keep0 = jnp.sum(row0 * (lanes == i % _LANE).astype(jnp.float32))
        row1 = kps[pl.ds(i1 // _LANE, 1), :]
        keep1 = jnp.sum(row1 * (lanes == i1 % _LANE).astype(jnp.float32))
        x10 = absm_ref[0, 0, i]
        y10 = absm_ref[0, 1, i]
        x20 = absm_ref[0, 2, i]
        y20 = absm_ref[0, 3, i]
        a0 = absm_ref[0, 4, i]
        x11 = absm_ref[0, 0, i1]
        y11 = absm_ref[0, 1, i1]
        x21 = absm_ref[0, 2, i1]
        y21 = absm_ref[0, 3, i1]
        a1 = absm_ref[0, 4, i1]
        alive0 = keep0 > 0.0
        sint = (jnp.maximum(jnp.minimum(x20, x21) - jnp.maximum(x10, x11),
                            0.0)
                * jnp.maximum(jnp.minimum(y20, y21) - jnp.maximum(y10, y11),
                              0.0))
        siou = sint / (a0 + a1 - sint + 1e-8)
        alive1 = (keep1 > 0.0) & jnp.logical_not(
            alive0 & (siou > _NMS_THRESH))
        it0 = (jnp.maximum(jnp.minimum(x2v, x20) - jnp.maximum(x1v, x10),
                           0.0)
               * jnp.maximum(jnp.minimum(y2v, y20) - jnp.maximum(y1v, y10),
                             0.0))
        iou0 = it0 / (a0 + arv - it0 + 1e-8)
        it1 = (jnp.maximum(jnp.minimum(x2v, x21) - jnp.maximum(x1v, x11),
                           0.0)
               * jnp.maximum(jnp.minimum(y2v, y21) - jnp.maximum(y1v, y11),
                             0.0))
        iou1 = it1 / (a1 + arv - it1 + 1e-8)
        sup = (((iou0 > _NMS_THRESH) & alive0)
               | ((iou1 > _NMS_THRESH) & alive1))
        kps[...] = jnp.where(sup, 0.0, kps[...])
        g0 = alive0 & (count < _NMS_POST)
        c1 = count + alive0.astype(jnp.int32)
        g1 = alive1 & (c1 < _NMS_POST)
        dstv = jnp.where((idx == i) & g0, count, dstv)
        dstv = jnp.where((idx == i1) & g1, c1, dstv)
        return i + 2, c1 + alive1.astype(jnp.int32), dstv

    _, _, dstv = jax.lax.while_loop(
        cond, body,
        (jnp.int32(0), jnp.int32(0),
         jnp.full((_SUB, _LANE), _NMS_POST, jnp.int32)))

    slot = jax.lax.broadcasted_iota(jnp.int32, (_NMS_POST, _LANE), 0)
    out = jnp.zeros((_NMS_POST, 16), jnp.float32)
    for k in range(_SUB):
        sel = (slot == jnp.broadcast_to(
            dstv[k:k + 1, :], (_NMS_POST, _LANE))).astype(jnp.float32)
        out = out + jnp.dot(sel, pk_ref[0, k * _LANE:(k + 1) * _LANE, :],
                            preferred_element_type=jnp.float32)
    rois_ref[0] = out[:, 0:7]
    scr_ref[0] = out[:, 7:8]
    labo_ref[0] = out[:, 11:12].astype(jnp.int32)
    lgo_ref[0] = out[:, 8:11]


@jax.jit
def _run(abv, absm, pk):
    B = abv.shape[0]
    return pl.pallas_call(
        _nms_body,
        grid=(B,),
        in_specs=[
            pl.BlockSpec((1, 5, _SUB, _LANE), lambda b: (b, 0, 0, 0)),
            pl.BlockSpec((1, 5, _NMS_PRE), lambda b: (b, 0, 0),
                         memory_space=pltpu.SMEM),
            pl.BlockSpec((1, _NMS_PRE, 16), lambda b: (b, 0, 0)),
        ],
        out_specs=[
            pl.BlockSpec((1, _NMS_POST, 7), lambda b: (b, 0, 0)),
            pl.BlockSpec((1, _NMS_POST, 1), lambda b: (b, 0, 0)),
            pl.BlockSpec((1, _NMS_POST, 1), lambda b: (b, 0, 0)),
            pl.BlockSpec((1, _NMS_POST, _NUM_CLASS), lambda b: (b, 0, 0)),
        ],
        out_shape=[
            jax.ShapeDtypeStruct((B, _NMS_POST, 7), jnp.float32),
            jax.ShapeDtypeStruct((B, _NMS_POST, 1), jnp.float32),
            jax.ShapeDtypeStruct((B, _NMS_POST, 1), jnp.int32),
            jax.ShapeDtypeStruct((B, _NMS_POST, _NUM_CLASS), jnp.float32),
        ],
        scratch_shapes=[pltpu.VMEM((_SUB, _LANE), jnp.float32)],
        compiler_params=pltpu.CompilerParams(
            dimension_semantics=("parallel",)),
    )(abv, absm, pk)


def kernel(batch_box_preds, batch_cls_preds, batch_size):
    scores = jnp.max(batch_cls_preds, axis=-1)
    _, order = jax.lax.top_k(scores, _NMS_PRE)
    payload = jnp.concatenate([batch_box_preds, batch_cls_preds], axis=-1)
    g = jnp.take_along_axis(payload, order[..., None], axis=1)
    b = g[..., 0:7]
    lg = g[..., 7:10]
    top_s = jnp.max(lg, axis=-1)
    lab = jnp.argmax(lg, axis=-1).astype(jnp.int32)
    B = b.shape[0]
    x = b[..., 0]
    y = b[..., 1]
    dx = b[..., 3]
    dy = b[..., 4]
    ry = b[..., 6]
    c = jnp.abs(jnp.cos(ry))
    s = jnp.abs(jnp.sin(ry))
    hw = 0.5 * (dx * c + dy * s)
    hh = 0.5 * (dx * s + dy * c)
    x1 = x - hw
    y1 = y - hh
    x2 = x + hw
    y2 = y + hh
    ab5 = jnp.stack([x1, y1, x2, y2, (x2 - x1) * (y2 - y1)], axis=1)
    abv = ab5.reshape(B, 5, _SUB, _LANE)
    pk = jnp.concatenate(
        [b, top_s[..., None], lg, (lab + 1).astype(jnp.float32)[..., None],
         jnp.zeros((B, _NMS_PRE, 4), jnp.float32)], axis=-1)
    rois, scr, labo, lgo = _run(abv, ab5, pk)
    return rois, scr[:, :, 0], labo[:, :, 0], lgo
